# baseline (speedup 1.0000x reference)
import functools

import numpy as np
import jax
from jax import lax
import jax.numpy as jnp
from jax.experimental import pallas as pl
from jax.experimental.pallas import tpu as pltpu
from jax.experimental.pallas import tpu_sc as plsc

_MAX_WAVELENGTH = 10000.0


def _sine_pe_np(seq_len: int, dim: int) -> np.ndarray:
    position = np.arange(seq_len, dtype=np.float64)
    min_freq = 1.0 / _MAX_WAVELENGTH
    timescales = np.power(
        min_freq,
        (2 * (np.arange(dim) // 2)).astype(np.float64) / float(dim),
    )
    angles = position[:, None] * timescales[None, :]
    cos_mask = (np.arange(dim) % 2).astype(np.float64)
    pe = np.sin(angles) * (1.0 - cos_mask) + np.cos(angles) * cos_mask
    return pe.astype(np.float32)


def _make_sc_kernel(batch, seq_len, dim, nb, nbuf=3):
    info = plsc.get_sparse_core_info()
    nc, ns, lanes = info.num_cores, info.num_subcores, info.num_lanes
    nw = nc * ns
    per_w = batch // nw
    n_chunks = per_w // nb
    mesh = plsc.VectorSubcoreMesh(core_axis_name="c", subcore_axis_name="s")

    @functools.partial(
        pl.kernel,
        mesh=mesh,
        out_type=jax.ShapeDtypeStruct((batch, seq_len, dim), jnp.float32),
        scratch_types=[
            pltpu.VMEM((nbuf, nb, seq_len, dim), jnp.float32),
            pltpu.VMEM((seq_len, dim), jnp.float32),
            pltpu.SemaphoreType.DMA((nbuf,)),
            pltpu.SemaphoreType.DMA((nbuf,)),
        ],
    )
    def sc_add(x_hbm, pe_hbm, out_hbm, buf, pe_v, sin, sout):
        wid = lax.axis_index("s") * nc + lax.axis_index("c")
        base = wid * per_w
        pltpu.sync_copy(pe_hbm, pe_v)

        def in_copy(k, slot):
            return pltpu.make_async_copy(
                x_hbm.at[pl.ds(base + k * nb, nb)], buf.at[slot], sin.at[slot])

        def out_copy(k, slot):
            return pltpu.make_async_copy(
                buf.at[slot], out_hbm.at[pl.ds(base + k * nb, nb)], sout.at[slot])

        in_copy(0, 0).start()
        in_copy(1, 1).start()

        def chunk(k, carry):
            slot = lax.rem(k, nbuf)
            in_copy(k, slot).wait()

            def si_body(si, c2):
                for j in range(dim // lanes):
                    pe_vec = pe_v[si, pl.ds(j * lanes, lanes)]
                    for b in range(nb):
                        buf[slot, b, si, pl.ds(j * lanes, lanes)] = (
                            buf[slot, b, si, pl.ds(j * lanes, lanes)] + pe_vec)
                return c2

            lax.fori_loop(0, seq_len, si_body, 0)
            out_copy(k, slot).start()

            @pl.when(k >= 1)
            def _():
                out_copy(k - 1, lax.rem(k - 1, nbuf)).wait()

            @pl.when(k + 2 < n_chunks)
            def _():
                in_copy(k + 2, lax.rem(k + 2, nbuf)).start()

            return carry

        lax.fori_loop(0, n_chunks, chunk, 0)
        out_copy(n_chunks - 1, lax.rem(n_chunks - 1, nbuf)).wait()

    return sc_add


def kernel(inputs, table):
    batch, seq_len, dim = inputs.shape
    pe = jnp.asarray(_sine_pe_np(seq_len, dim))
    return _make_sc_kernel(batch, seq_len, dim, nb=4)(inputs, pe)


# speedup vs baseline: 1.0837x; 1.0837x over previous
import functools

import numpy as np
import jax
from jax import lax
import jax.numpy as jnp
from jax.experimental import pallas as pl
from jax.experimental.pallas import tpu as pltpu
from jax.experimental.pallas import tpu_sc as plsc

_MAX_WAVELENGTH = 10000.0


def _sine_pe_np(seq_len: int, dim: int) -> np.ndarray:
    position = np.arange(seq_len, dtype=np.float64)
    min_freq = 1.0 / _MAX_WAVELENGTH
    timescales = np.power(
        min_freq,
        (2 * (np.arange(dim) // 2)).astype(np.float64) / float(dim),
    )
    angles = position[:, None] * timescales[None, :]
    cos_mask = (np.arange(dim) % 2).astype(np.float64)
    pe = np.sin(angles) * (1.0 - cos_mask) + np.cos(angles) * cos_mask
    return pe.astype(np.float32)


def _make_sc_kernel(batch, seq_len, dim, nb, nbuf=3):
    info = plsc.get_sparse_core_info()
    nc, ns, lanes = info.num_cores, info.num_subcores, info.num_lanes
    nw = nc * ns
    per_w = batch // nw
    n_chunks = per_w // nb
    mesh = plsc.VectorSubcoreMesh(core_axis_name="c", subcore_axis_name="s")

    @functools.partial(
        pl.kernel,
        mesh=mesh,
        out_type=jax.ShapeDtypeStruct((batch, seq_len, dim), jnp.float32),
        scratch_types=[
            pltpu.VMEM((nbuf, nb, seq_len, dim), jnp.float32),
            pltpu.VMEM((seq_len, dim), jnp.float32),
            pltpu.SemaphoreType.DMA((nbuf,)),
            pltpu.SemaphoreType.DMA((nbuf,)),
        ],
    )
    def sc_add(x_hbm, pe_hbm, out_hbm, buf, pe_v, sin, sout):
        wid = lax.axis_index("s") * nc + lax.axis_index("c")
        base = wid * per_w
        pltpu.sync_copy(pe_hbm, pe_v)

        def in_copy(k, slot):
            return pltpu.make_async_copy(
                x_hbm.at[pl.ds(base + k * nb, nb)], buf.at[slot], sin.at[slot])

        def out_copy(k, slot):
            return pltpu.make_async_copy(
                buf.at[slot], out_hbm.at[pl.ds(base + k * nb, nb)], sout.at[slot])

        in_copy(0, 0).start()
        in_copy(1, 1).start()

        def chunk(k, carry):
            slot = lax.rem(k, nbuf)
            in_copy(k, slot).wait()

            for si in range(seq_len):
                for j in range(dim // lanes):
                    pe_vec = pe_v[si, pl.ds(j * lanes, lanes)]
                    for b in range(nb):
                        buf[slot, b, si, pl.ds(j * lanes, lanes)] = (
                            buf[slot, b, si, pl.ds(j * lanes, lanes)] + pe_vec)
            out_copy(k, slot).start()

            @pl.when(k >= 1)
            def _():
                out_copy(k - 1, lax.rem(k - 1, nbuf)).wait()

            @pl.when(k + 2 < n_chunks)
            def _():
                in_copy(k + 2, lax.rem(k + 2, nbuf)).start()

            return carry

        lax.fori_loop(0, n_chunks, chunk, 0)
        out_copy(n_chunks - 1, lax.rem(n_chunks - 1, nbuf)).wait()

    return sc_add


def kernel(inputs, table):
    batch, seq_len, dim = inputs.shape
    pe = jnp.asarray(_sine_pe_np(seq_len, dim))
    return _make_sc_kernel(batch, seq_len, dim, nb=4)(inputs, pe)


# speedup vs baseline: 1.2973x; 1.1971x over previous
import functools

import numpy as np
import jax
from jax import lax
import jax.numpy as jnp
from jax.experimental import pallas as pl
from jax.experimental.pallas import tpu as pltpu
from jax.experimental.pallas import tpu_sc as plsc

_MAX_WAVELENGTH = 10000.0


def _sine_pe_np(seq_len: int, dim: int) -> np.ndarray:
    position = np.arange(seq_len, dtype=np.float64)
    min_freq = 1.0 / _MAX_WAVELENGTH
    timescales = np.power(
        min_freq,
        (2 * (np.arange(dim) // 2)).astype(np.float64) / float(dim),
    )
    angles = position[:, None] * timescales[None, :]
    cos_mask = (np.arange(dim) % 2).astype(np.float64)
    pe = np.sin(angles) * (1.0 - cos_mask) + np.cos(angles) * cos_mask
    return pe.astype(np.float32)


def _make_sc_kernel(batch, batch0, seq_len, dim, nb, nbuf=3):
    info = plsc.get_sparse_core_info()
    nc, ns, lanes = info.num_cores, info.num_subcores, info.num_lanes
    nw = nc * ns
    sc_batch = batch - batch0
    per_w = sc_batch // nw
    n_chunks = per_w // nb
    mesh = plsc.VectorSubcoreMesh(core_axis_name="c", subcore_axis_name="s")

    @functools.partial(
        pl.kernel,
        mesh=mesh,
        out_type=jax.ShapeDtypeStruct((sc_batch, seq_len, dim), jnp.float32),
        scratch_types=[
            pltpu.VMEM((nbuf, nb, seq_len, dim), jnp.float32),
            pltpu.VMEM((seq_len, dim), jnp.float32),
            pltpu.SemaphoreType.DMA((nbuf,)),
            pltpu.SemaphoreType.DMA((nbuf,)),
        ],
    )
    def sc_add(x_hbm, pe_hbm, out_hbm, buf, pe_v, sin, sout):
        wid = lax.axis_index("s") * nc + lax.axis_index("c")
        rbase = batch0 + wid * per_w
        wbase = wid * per_w
        pltpu.sync_copy(pe_hbm, pe_v)

        def in_copy(k, slot):
            return pltpu.make_async_copy(
                x_hbm.at[pl.ds(rbase + k * nb, nb)], buf.at[slot], sin.at[slot])

        def out_copy(k, slot):
            return pltpu.make_async_copy(
                buf.at[slot], out_hbm.at[pl.ds(wbase + k * nb, nb)],
                sout.at[slot])

        in_copy(0, 0).start()
        in_copy(1, 1).start()

        def chunk(k, carry):
            slot = lax.rem(k, nbuf)
            in_copy(k, slot).wait()
            for si in range(seq_len):
                for j in range(dim // lanes):
                    pe_vec = pe_v[si, pl.ds(j * lanes, lanes)]
                    for b in range(nb):
                        buf[slot, b, si, pl.ds(j * lanes, lanes)] = (
                            buf[slot, b, si, pl.ds(j * lanes, lanes)] + pe_vec)
            out_copy(k, slot).start()

            @pl.when(k >= 1)
            def _():
                out_copy(k - 1, lax.rem(k - 1, nbuf)).wait()

            @pl.when(k + 2 < n_chunks)
            def _():
                in_copy(k + 2, lax.rem(k + 2, nbuf)).start()

            return carry

        lax.fori_loop(0, n_chunks, chunk, 0)
        out_copy(n_chunks - 1, lax.rem(n_chunks - 1, nbuf)).wait()

    return sc_add


def _make_tc_body(bb, nbuf, seq_len, dim):
    def body(x_hbm, pe_ref, o_hbm, bin_ref, bout_ref, sin, sout):
        i = pl.program_id(0)
        n = pl.num_programs(0)

        def in_copy(j, slot):
            return pltpu.make_async_copy(
                x_hbm.at[pl.ds(j * bb, bb)], bin_ref.at[slot], sin.at[slot])

        def out_copy(j, slot):
            return pltpu.make_async_copy(
                bout_ref.at[slot], o_hbm.at[pl.ds(j * bb, bb)], sout.at[slot])

        slot = lax.rem(i, nbuf)

        @pl.when(i == 0)
        def _():
            for s in range(nbuf - 1):
                in_copy(s, s).start()

        nxt = i + nbuf - 1

        @pl.when(nxt < n)
        def _():
            in_copy(nxt, lax.rem(nxt, nbuf)).start()

        in_copy(i, slot).wait()

        @pl.when(i >= nbuf)
        def _():
            out_copy(i - nbuf, slot).wait()

        bout_ref[slot] = bin_ref[slot] + pe_ref[...]
        out_copy(i, slot).start()

        @pl.when(i == n - 1)
        def _():
            for k in range(nbuf):
                j = n - nbuf + k
                out_copy(j, lax.rem(j, nbuf)).wait()

    return body


def _tc_add(inputs, pe, tc_batch, bb, nbuf=3):
    batch, seq_len, dim = inputs.shape
    grid = (tc_batch // bb,)
    return pl.pallas_call(
        _make_tc_body(bb, nbuf, seq_len, dim),
        grid=grid,
        in_specs=[
            pl.BlockSpec(memory_space=pl.ANY),
            pl.BlockSpec((1, seq_len, dim), lambda i: (0, 0, 0)),
        ],
        out_specs=pl.BlockSpec(memory_space=pl.ANY),
        out_shape=jax.ShapeDtypeStruct((batch, seq_len, dim), jnp.float32),
        scratch_shapes=[
            pltpu.VMEM((nbuf, bb, seq_len, dim), jnp.float32),
            pltpu.VMEM((nbuf, bb, seq_len, dim), jnp.float32),
            pltpu.SemaphoreType.DMA((nbuf,)),
            pltpu.SemaphoreType.DMA((nbuf,)),
        ],
    )(inputs, pe)


def kernel(inputs, table):
    batch, seq_len, dim = inputs.shape
    pe_rows = _sine_pe_np(seq_len, dim)
    pe = jnp.asarray(pe_rows)

    tc_batch = 2560
    sc_out = _make_sc_kernel(batch, tc_batch, seq_len, dim, nb=4)(inputs, pe)
    tc_out = _tc_add(inputs, pe[None], tc_batch, bb=128)
    return lax.dynamic_update_slice(tc_out, sc_out, (tc_batch, 0, 0))


# speedup vs baseline: 1.3543x; 1.0439x over previous
import functools

import numpy as np
import jax
from jax import lax
import jax.numpy as jnp
from jax.experimental import pallas as pl
from jax.experimental.pallas import tpu as pltpu
from jax.experimental.pallas import tpu_sc as plsc

_MAX_WAVELENGTH = 10000.0


def _sine_pe_np(seq_len: int, dim: int) -> np.ndarray:
    position = np.arange(seq_len, dtype=np.float64)
    min_freq = 1.0 / _MAX_WAVELENGTH
    timescales = np.power(
        min_freq,
        (2 * (np.arange(dim) // 2)).astype(np.float64) / float(dim),
    )
    angles = position[:, None] * timescales[None, :]
    cos_mask = (np.arange(dim) % 2).astype(np.float64)
    pe = np.sin(angles) * (1.0 - cos_mask) + np.cos(angles) * cos_mask
    return pe.astype(np.float32)


def _make_sc_kernel(batch, batch0, seq_len, dim, nb, nbuf=3):
    info = plsc.get_sparse_core_info()
    nc, ns, lanes = info.num_cores, info.num_subcores, info.num_lanes
    nw = nc * ns
    sc_batch = batch - batch0
    per_w = sc_batch // nw
    n_chunks = per_w // nb
    mesh = plsc.VectorSubcoreMesh(core_axis_name="c", subcore_axis_name="s")

    @functools.partial(
        pl.kernel,
        mesh=mesh,
        out_type=jax.ShapeDtypeStruct((sc_batch, seq_len, dim), jnp.float32),
        scratch_types=[
            pltpu.VMEM((nbuf, nb, seq_len, dim), jnp.float32),
            pltpu.VMEM((seq_len, dim), jnp.float32),
            pltpu.SemaphoreType.DMA((nbuf,)),
            pltpu.SemaphoreType.DMA((nbuf,)),
        ],
    )
    def sc_add(x_hbm, pe_hbm, out_hbm, buf, pe_v, sin, sout):
        wid = lax.axis_index("s") * nc + lax.axis_index("c")
        rbase = batch0 + wid * per_w
        wbase = wid * per_w
        pltpu.sync_copy(pe_hbm, pe_v)

        def in_copy(k, slot):
            return pltpu.make_async_copy(
                x_hbm.at[pl.ds(rbase + k * nb, nb)], buf.at[slot], sin.at[slot])

        def out_copy(k, slot):
            return pltpu.make_async_copy(
                buf.at[slot], out_hbm.at[pl.ds(wbase + k * nb, nb)],
                sout.at[slot])

        in_copy(0, 0).start()
        in_copy(1, 1).start()

        def chunk(k, carry):
            slot = lax.rem(k, nbuf)
            in_copy(k, slot).wait()
            for si in range(seq_len):
                for j in range(dim // lanes):
                    pe_vec = pe_v[si, pl.ds(j * lanes, lanes)]
                    for b in range(nb):
                        buf[slot, b, si, pl.ds(j * lanes, lanes)] = (
                            buf[slot, b, si, pl.ds(j * lanes, lanes)] + pe_vec)
            out_copy(k, slot).start()

            @pl.when(k >= 1)
            def _():
                out_copy(k - 1, lax.rem(k - 1, nbuf)).wait()

            @pl.when(k + 2 < n_chunks)
            def _():
                in_copy(k + 2, lax.rem(k + 2, nbuf)).start()

            return carry

        lax.fori_loop(0, n_chunks, chunk, 0)
        out_copy(n_chunks - 1, lax.rem(n_chunks - 1, nbuf)).wait()

    return sc_add


def _make_tc_body(bb, nbuf, seq_len, dim):
    def body(x_hbm, pe_ref, o_hbm, bin_ref, bout_ref, sin, sout):
        i = pl.program_id(0)
        n = pl.num_programs(0)

        def in_copy(j, slot):
            return pltpu.make_async_copy(
                x_hbm.at[pl.ds(j * bb, bb)], bin_ref.at[slot], sin.at[slot])

        def out_copy(j, slot):
            return pltpu.make_async_copy(
                bout_ref.at[slot], o_hbm.at[pl.ds(j * bb, bb)], sout.at[slot])

        slot = lax.rem(i, nbuf)

        @pl.when(i == 0)
        def _():
            for s in range(nbuf - 1):
                in_copy(s, s).start()

        nxt = i + nbuf - 1

        @pl.when(nxt < n)
        def _():
            in_copy(nxt, lax.rem(nxt, nbuf)).start()

        in_copy(i, slot).wait()

        @pl.when(i >= nbuf)
        def _():
            out_copy(i - nbuf, slot).wait()

        bout_ref[slot] = bin_ref[slot] + pe_ref[...]
        out_copy(i, slot).start()

        @pl.when(i == n - 1)
        def _():
            for k in range(nbuf):
                j = n - nbuf + k
                out_copy(j, lax.rem(j, nbuf)).wait()

    return body


def _tc_add(inputs, pe, tc_batch, bb, nbuf=3):
    batch, seq_len, dim = inputs.shape
    grid = (tc_batch // bb,)
    return pl.pallas_call(
        _make_tc_body(bb, nbuf, seq_len, dim),
        grid=grid,
        in_specs=[
            pl.BlockSpec(memory_space=pl.ANY),
            pl.BlockSpec((1, seq_len, dim), lambda i: (0, 0, 0)),
        ],
        out_specs=pl.BlockSpec(memory_space=pl.ANY),
        out_shape=jax.ShapeDtypeStruct((batch, seq_len, dim), jnp.float32),
        scratch_shapes=[
            pltpu.VMEM((nbuf, bb, seq_len, dim), jnp.float32),
            pltpu.VMEM((nbuf, bb, seq_len, dim), jnp.float32),
            pltpu.SemaphoreType.DMA((nbuf,)),
            pltpu.SemaphoreType.DMA((nbuf,)),
        ],
    )(inputs, pe)


def kernel(inputs, table):
    batch, seq_len, dim = inputs.shape
    pe_rows = _sine_pe_np(seq_len, dim)
    pe = jnp.asarray(pe_rows)

    tc_batch = 2816
    sc_out = _make_sc_kernel(batch, tc_batch, seq_len, dim, nb=4)(inputs, pe)
    tc_out = _tc_add(inputs, pe[None], tc_batch, bb=128)
    one = inputs[0, 0, 0] * 0.0 + 1.0
    return lax.dynamic_update_slice(tc_out, sc_out * one, (tc_batch, 0, 0))


# speedup vs baseline: 1.3775x; 1.0172x over previous
import functools

import numpy as np
import jax
from jax import lax
import jax.numpy as jnp
from jax.experimental import pallas as pl
from jax.experimental.pallas import tpu as pltpu
from jax.experimental.pallas import tpu_sc as plsc

_MAX_WAVELENGTH = 10000.0


def _sine_pe_np(seq_len: int, dim: int) -> np.ndarray:
    position = np.arange(seq_len, dtype=np.float64)
    min_freq = 1.0 / _MAX_WAVELENGTH
    timescales = np.power(
        min_freq,
        (2 * (np.arange(dim) // 2)).astype(np.float64) / float(dim),
    )
    angles = position[:, None] * timescales[None, :]
    cos_mask = (np.arange(dim) % 2).astype(np.float64)
    pe = np.sin(angles) * (1.0 - cos_mask) + np.cos(angles) * cos_mask
    return pe.astype(np.float32)


def _make_sc_kernel(batch, batch0, seq_len, dim, nb, nbuf=3):
    info = plsc.get_sparse_core_info()
    nc, ns, lanes = info.num_cores, info.num_subcores, info.num_lanes
    nw = nc * ns
    sc_batch = batch - batch0
    per_w = sc_batch // nw
    n_chunks = per_w // nb
    mesh = plsc.VectorSubcoreMesh(core_axis_name="c", subcore_axis_name="s")

    @functools.partial(
        pl.kernel,
        mesh=mesh,
        out_type=jax.ShapeDtypeStruct((sc_batch, seq_len, dim), jnp.float32),
        scratch_types=[
            pltpu.VMEM((nbuf, nb, seq_len, dim), jnp.float32),
            pltpu.VMEM((seq_len, dim), jnp.float32),
            pltpu.SemaphoreType.DMA((nbuf,)),
            pltpu.SemaphoreType.DMA((nbuf,)),
        ],
    )
    def sc_add(x_hbm, pe_hbm, out_hbm, buf, pe_v, sin, sout):
        wid = lax.axis_index("s") * nc + lax.axis_index("c")
        rbase = batch0 + wid * per_w
        wbase = wid * per_w
        pltpu.sync_copy(pe_hbm, pe_v)

        def in_copy(k, slot):
            return pltpu.make_async_copy(
                x_hbm.at[pl.ds(rbase + k * nb, nb)], buf.at[slot], sin.at[slot])

        def out_copy(k, slot):
            return pltpu.make_async_copy(
                buf.at[slot], out_hbm.at[pl.ds(wbase + k * nb, nb)],
                sout.at[slot])

        in_copy(0, 0).start()
        in_copy(1, 1).start()

        def chunk(k, carry):
            slot = lax.rem(k, nbuf)
            in_copy(k, slot).wait()
            for si in range(seq_len):
                for j in range(dim // lanes):
                    pe_vec = pe_v[si, pl.ds(j * lanes, lanes)]
                    for b in range(nb):
                        buf[slot, b, si, pl.ds(j * lanes, lanes)] = (
                            buf[slot, b, si, pl.ds(j * lanes, lanes)] + pe_vec)
            out_copy(k, slot).start()

            @pl.when(k >= 1)
            def _():
                out_copy(k - 1, lax.rem(k - 1, nbuf)).wait()

            @pl.when(k + 2 < n_chunks)
            def _():
                in_copy(k + 2, lax.rem(k + 2, nbuf)).start()

            return carry

        lax.fori_loop(0, n_chunks, chunk, 0)
        out_copy(n_chunks - 1, lax.rem(n_chunks - 1, nbuf)).wait()

    return sc_add


def _make_tc_body(bb, nbuf, seq_len, dim):
    def body(x_hbm, pe_ref, o_hbm, bin_ref, bout_ref, sin, sout):
        i = pl.program_id(0)
        n = pl.num_programs(0)

        def in_copy(j, slot):
            return pltpu.make_async_copy(
                x_hbm.at[pl.ds(j * bb, bb)], bin_ref.at[slot], sin.at[slot])

        def out_copy(j, slot):
            return pltpu.make_async_copy(
                bout_ref.at[slot], o_hbm.at[pl.ds(j * bb, bb)], sout.at[slot])

        slot = lax.rem(i, nbuf)

        @pl.when(i == 0)
        def _():
            for s in range(nbuf - 1):
                in_copy(s, s).start()

        nxt = i + nbuf - 1

        @pl.when(nxt < n)
        def _():
            in_copy(nxt, lax.rem(nxt, nbuf)).start()

        in_copy(i, slot).wait()

        @pl.when(i >= nbuf)
        def _():
            out_copy(i - nbuf, slot).wait()

        bout_ref[slot] = bin_ref[slot] + pe_ref[...]
        out_copy(i, slot).start()

        @pl.when(i == n - 1)
        def _():
            for k in range(nbuf):
                j = n - nbuf + k
                out_copy(j, lax.rem(j, nbuf)).wait()

    return body


def _tc_add(inputs, pe, tc_batch, bb, nbuf=3):
    batch, seq_len, dim = inputs.shape
    grid = (tc_batch // bb,)
    return pl.pallas_call(
        _make_tc_body(bb, nbuf, seq_len, dim),
        grid=grid,
        in_specs=[
            pl.BlockSpec(memory_space=pl.ANY),
            pl.BlockSpec((1, seq_len, dim), lambda i: (0, 0, 0)),
        ],
        out_specs=pl.BlockSpec(memory_space=pl.ANY),
        out_shape=jax.ShapeDtypeStruct((batch, seq_len, dim), jnp.float32),
        scratch_shapes=[
            pltpu.VMEM((nbuf, bb, seq_len, dim), jnp.float32),
            pltpu.VMEM((nbuf, bb, seq_len, dim), jnp.float32),
            pltpu.SemaphoreType.DMA((nbuf,)),
            pltpu.SemaphoreType.DMA((nbuf,)),
        ],
    )(inputs, pe)


def kernel(inputs, table):
    batch, seq_len, dim = inputs.shape
    pe_rows = _sine_pe_np(seq_len, dim)
    pe = jnp.asarray(pe_rows)

    tc_batch = 3072
    sc_out = _make_sc_kernel(batch, tc_batch, seq_len, dim, nb=4)(inputs, pe)
    tc_out = _tc_add(inputs, pe[None], tc_batch, bb=128)
    one = inputs[0, 0, 0] * 0.0 + 1.0
    return lax.dynamic_update_slice(tc_out, sc_out * one, (tc_batch, 0, 0))
